# baseline (speedup 1.0000x reference)
import functools

import jax
import jax.numpy as jnp
import numpy as _np
from jax import lax
from jax.experimental import pallas as pl
from jax.experimental.pallas import tpu as pltpu
from jax.experimental.pallas import tpu_sc as plsc

N_NODES = 10000
N_EDGES = 320000
D = 128
NC, NS, L = 2, 16, 16
NW = NC * NS
EPW = N_EDGES // NW
CHUNK = 80
NCHUNK = EPW // CHUNK
NGROUP = NCHUNK // 2

_mesh = plsc.VectorSubcoreMesh(
    core_axis_name="c", subcore_axis_name="s", num_cores=NC, num_subcores=NS
)


_SRC_PERM = (
    _np.arange(D).reshape(D // (2 * L), 2, L).transpose(0, 2, 1).reshape(D)
)


def _mm_body(x_ref, w_ref, b_ref, y1_ref, y2_ref, y1b_ref, y2b_ref):
    xb = x_ref[...]
    w = w_ref[...]
    dn = (((1,), (1,)), ((), ()))
    y1 = (
        lax.dot_general(xb, w[:, :D], dn, preferred_element_type=jnp.float32)
        + b_ref[...]
    )
    y2 = lax.dot_general(
        xb, w[:, D:], dn, preferred_element_type=jnp.float32
    )
    y1_ref[...] = y1
    y2_ref[...] = y2
    y1b_ref[...] = y1.astype(jnp.bfloat16)
    y2b_ref[...] = y2.astype(jnp.bfloat16)


def _matmuls(x, W, b):
    return pl.pallas_call(
        _mm_body,
        out_shape=(
            jax.ShapeDtypeStruct((N_NODES, D), jnp.float32),
            jax.ShapeDtypeStruct((N_NODES, D), jnp.float32),
            jax.ShapeDtypeStruct((N_NODES, D), jnp.bfloat16),
            jax.ShapeDtypeStruct((N_NODES, D), jnp.bfloat16),
        ),
    )(x, W, b.reshape(1, D))


def _acc_body(c1_hbm, c2_hbm, src_hbm, dst_hbm, acc_hbm,
              c1_v, c2_v, src_v, dst_v, vals_v, zero_v, acc_sh):
    cid = lax.axis_index("c")
    sid = lax.axis_index("s")
    wid = cid * NS + sid
    base = wid * EPW

    @pl.when(sid == 0)
    def _zero():
        def zloop(i, carry):
            zero_v[pl.ds(i * L, L)] = jnp.zeros((L,), jnp.float32)
            return carry
        lax.fori_loop(0, N_NODES // L, zloop, 0)
        pltpu.sync_copy(zero_v, acc_sh)

    pltpu.sync_copy(c1_hbm, c1_v)
    pltpu.sync_copy(c2_hbm, c2_v)
    pltpu.sync_copy(src_hbm.at[pl.ds(base, EPW)], src_v)
    pltpu.sync_copy(dst_hbm.at[pl.ds(base, EPW)], dst_v)

    def body(i, carry):
        s_idx = src_v[pl.ds(i * L, L)]
        d_idx = dst_v[pl.ds(i * L, L)]
        v1 = plsc.load_gather(c1_v, [s_idx])
        v2 = plsc.load_gather(c2_v, [d_idx])
        vals_v[pl.ds(i * L, L)] = jnp.maximum(v1 + v2, 0.0)
        return carry
    lax.fori_loop(0, EPW // L, body, 0)

    plsc.subcore_barrier()
    pltpu.sync_copy(vals_v, acc_sh.at[src_v], add=True)
    plsc.subcore_barrier()

    @pl.when(sid == 0)
    def _writeout():
        pltpu.sync_copy(acc_sh, acc_hbm.at[cid])


_acc_call = pl.kernel(
    _acc_body,
    out_type=jax.ShapeDtypeStruct((NC, N_NODES), jnp.float32),
    mesh=_mesh,
    scratch_types=[
        pltpu.VMEM((N_NODES,), jnp.float32),
        pltpu.VMEM((N_NODES,), jnp.float32),
        pltpu.VMEM((EPW,), jnp.int32),
        pltpu.VMEM((EPW,), jnp.int32),
        pltpu.VMEM((EPW,), jnp.float32),
        pltpu.VMEM((N_NODES,), jnp.float32),
        pltpu.VMEM_SHARED((N_NODES,), jnp.float32),
    ],
    compiler_params=pltpu.CompilerParams(needs_layout_passes=False),
)


def _edge_body(y1_hbm, y2_hbm, src_hbm, dst_hbm, acc2_hbm, out_hbm,
               src_v, dst_v, acc_v, acc2_v,
               r1a, r2a, oa, r1b, r2b, ob,
               semg_a, semg_b, semw_a, semw_b):
    cid = lax.axis_index("c")
    sid = lax.axis_index("s")
    wid = cid * NS + sid
    base = wid * EPW

    pltpu.sync_copy(src_hbm.at[pl.ds(base, EPW)], src_v)
    pltpu.sync_copy(dst_hbm.at[pl.ds(base, EPW)], dst_v)

    @pl.when(wid == 0)
    def _stage_acc():
        pltpu.sync_copy(acc2_hbm.at[0], acc_v.at[pl.ds(0, N_NODES)])
        pltpu.sync_copy(acc2_hbm.at[1], acc2_v)

        def addloop(i, carry):
            sl = pl.ds(i * L, L)
            acc_v[sl] = acc_v[sl] + acc2_v[sl]
            return carry
        lax.fori_loop(0, N_NODES // L, addloop, 0)

    slots = ((r1a, r2a, oa, semg_a, semw_a),
             (r1b, r2b, ob, semg_b, semw_b))

    def issue_gather(c, slot):
        r1, r2, _, semg, _ = slot
        off = c * CHUNK
        pltpu.async_copy(y1_hbm.at[src_v.at[pl.ds(off, CHUNK)]], r1, semg)
        pltpu.async_copy(y2_hbm.at[dst_v.at[pl.ds(off, CHUNK)]], r2, semg)

    def do_chunk(c, slot, nslot, issue_next, drain_write):
        r1, r2, o, semg, semw = slot
        if issue_next:
            issue_gather(c + 1, nslot)

        off = c * CHUNK
        pltpu.make_async_copy(
            y1_hbm.at[src_v.at[pl.ds(off, CHUNK)]], r1, semg).wait()
        pltpu.make_async_copy(
            y2_hbm.at[dst_v.at[pl.ds(off, CHUNK)]], r2, semg).wait()

        if drain_write:
            pltpu.make_async_copy(o, out_hbm.at[pl.ds(0, CHUNK)], semw).wait()

        @plsc.parallel_loop(0, CHUNK, unroll=4)
        def row_body(i):
            for k in range(D // (2 * L)):
                w1 = plsc.bitcast(r1[i, pl.ds(k * L, L)], jnp.bfloat16)
                w2 = plsc.bitcast(r2[i, pl.ds(k * L, L)], jnp.bfloat16)
                a1, b1 = plsc.unpack(w1, format=plsc.PackFormat.INTERLEAVED)
                a2, b2 = plsc.unpack(w2, format=plsc.PackFormat.INTERLEAVED)
                o[i, pl.ds(k * 2 * L, L)] = jnp.maximum(a1 + a2, 0.0)
                o[i, pl.ds(k * 2 * L + L, L)] = jnp.maximum(b1 + b2, 0.0)

        @pl.when(wid == 0)
        def _fix_col0():
            for j in range((CHUNK + L - 1) // L):
                valid = min(CHUNK - j * L, L)
                rows = lax.iota(jnp.int32, L) + j * L
                vals = acc_v[pl.ds(c * CHUNK + j * L, L)]
                if valid == L:
                    mask = None
                else:
                    mask = (lax.iota(jnp.int32, L)
                            < jnp.full((L,), valid, jnp.int32))
                plsc.addupdate_scatter(
                    o, [rows, jnp.zeros((L,), jnp.int32)], vals, mask=mask
                )

        pltpu.async_copy(o, out_hbm.at[pl.ds(base + c * CHUNK, CHUNK)], semw)

    issue_gather(0, slots[0])
    do_chunk(0, slots[0], slots[1], True, False)
    do_chunk(1, slots[1], slots[0], True, False)

    def group(g, carry):
        do_chunk(2 * g, slots[0], slots[1], True, True)
        do_chunk(2 * g + 1, slots[1], slots[0], True, True)
        return carry
    lax.fori_loop(1, (NCHUNK - 1) // 2, group, 0)

    do_chunk(NCHUNK - 1, slots[0], slots[1], False, True)

    pltpu.make_async_copy(oa, out_hbm.at[pl.ds(0, CHUNK)], semw_a).wait()
    pltpu.make_async_copy(ob, out_hbm.at[pl.ds(0, CHUNK)], semw_b).wait()


_edge_call = pl.kernel(
    _edge_body,
    out_type=jax.ShapeDtypeStruct((N_EDGES, D), jnp.float32),
    mesh=_mesh,
    scratch_types=[
        pltpu.VMEM((EPW,), jnp.int32),
        pltpu.VMEM((EPW,), jnp.int32),
        pltpu.VMEM((N_NODES + L,), jnp.float32),
        pltpu.VMEM((N_NODES,), jnp.float32),
        pltpu.VMEM((CHUNK, D // 2), jnp.int32),
        pltpu.VMEM((CHUNK, D // 2), jnp.int32),
        pltpu.VMEM((CHUNK, D), jnp.float32),
        pltpu.VMEM((CHUNK, D // 2), jnp.int32),
        pltpu.VMEM((CHUNK, D // 2), jnp.int32),
        pltpu.VMEM((CHUNK, D), jnp.float32),
        pltpu.SemaphoreType.DMA,
        pltpu.SemaphoreType.DMA,
        pltpu.SemaphoreType.DMA,
        pltpu.SemaphoreType.DMA,
    ],
    compiler_params=pltpu.CompilerParams(
        needs_layout_passes=False, use_tc_tiling_on_sc=False
    ),
)


def _words(yb):
    return lax.bitcast_convert_type(
        yb.reshape(N_NODES, D // 2, 2), jnp.int32
    )


def kernel(x, edge_index, W, b):
    src = edge_index[0]
    dst = edge_index[1]
    y1, y2, y1b, y2b = _matmuls(x, W[_SRC_PERM], b[_SRC_PERM])
    c1 = y1[:, 0]
    c2 = y2[:, 0]
    acc2 = _acc_call(c1, c2, src, dst)
    return _edge_call(_words(y1b), _words(y2b), src, dst, acc2)


# speedup vs baseline: 1.1829x; 1.1829x over previous
import functools

import jax
import jax.numpy as jnp
from jax import lax
from jax.experimental import pallas as pl
from jax.experimental.pallas import tpu as pltpu
from jax.experimental.pallas import tpu_sc as plsc

N_NODES = 10000
N_EDGES = 320000
D = 128
NC, NS, L = 2, 16, 16
NW = NC * NS
EPW = N_EDGES // NW
CHUNK = 80
NCHUNK = EPW // CHUNK
NGROUP = NCHUNK // 2

_mesh = plsc.VectorSubcoreMesh(
    core_axis_name="c", subcore_axis_name="s", num_cores=NC, num_subcores=NS
)


def _mm_body(x_ref, w_ref, b_ref, y1_ref, y2_ref):
    xb = x_ref[...]
    w = w_ref[...]
    dn = (((1,), (1,)), ((), ()))
    y1_ref[...] = (
        lax.dot_general(xb, w[:, :D], dn, preferred_element_type=jnp.float32)
        + b_ref[...]
    )
    y2_ref[...] = lax.dot_general(
        xb, w[:, D:], dn, preferred_element_type=jnp.float32
    )


def _matmuls(x, W, b):
    return pl.pallas_call(
        _mm_body,
        out_shape=(
            jax.ShapeDtypeStruct((N_NODES, D), jnp.float32),
            jax.ShapeDtypeStruct((N_NODES, D), jnp.float32),
        ),
    )(x, W, b.reshape(1, D))


def _acc_body(c1_hbm, c2_hbm, src_hbm, dst_hbm, acc_hbm,
              c1_v, c2_v, src_v, dst_v, vals_v, zero_v, acc_sh):
    cid = lax.axis_index("c")
    sid = lax.axis_index("s")
    wid = cid * NS + sid
    base = wid * EPW

    @pl.when(sid == 0)
    def _zero():
        def zloop(i, carry):
            zero_v[pl.ds(i * L, L)] = jnp.zeros((L,), jnp.float32)
            return carry
        lax.fori_loop(0, N_NODES // L, zloop, 0)
        pltpu.sync_copy(zero_v, acc_sh)

    pltpu.sync_copy(c1_hbm, c1_v)
    pltpu.sync_copy(c2_hbm, c2_v)
    pltpu.sync_copy(src_hbm.at[pl.ds(base, EPW)], src_v)
    pltpu.sync_copy(dst_hbm.at[pl.ds(base, EPW)], dst_v)

    def body(i, carry):
        s_idx = src_v[pl.ds(i * L, L)]
        d_idx = dst_v[pl.ds(i * L, L)]
        v1 = plsc.load_gather(c1_v, [s_idx])
        v2 = plsc.load_gather(c2_v, [d_idx])
        vals_v[pl.ds(i * L, L)] = jnp.maximum(v1 + v2, 0.0)
        return carry
    lax.fori_loop(0, EPW // L, body, 0)

    plsc.subcore_barrier()
    pltpu.sync_copy(vals_v, acc_sh.at[src_v], add=True)
    plsc.subcore_barrier()

    @pl.when(sid == 0)
    def _writeout():
        pltpu.sync_copy(acc_sh, acc_hbm.at[cid])


_acc_call = pl.kernel(
    _acc_body,
    out_type=jax.ShapeDtypeStruct((NC, N_NODES), jnp.float32),
    mesh=_mesh,
    scratch_types=[
        pltpu.VMEM((N_NODES,), jnp.float32),
        pltpu.VMEM((N_NODES,), jnp.float32),
        pltpu.VMEM((EPW,), jnp.int32),
        pltpu.VMEM((EPW,), jnp.int32),
        pltpu.VMEM((EPW,), jnp.float32),
        pltpu.VMEM((N_NODES,), jnp.float32),
        pltpu.VMEM_SHARED((N_NODES,), jnp.float32),
    ],
    compiler_params=pltpu.CompilerParams(needs_layout_passes=False),
)


def _edge_body(y1_hbm, y2_hbm, src_hbm, dst_hbm, accsum_hbm, out_hbm,
               src_v, dst_v, acc_v,
               r1a, r2a, oa, r1b, r2b, ob,
               semg_a, semg_b, semw_a, semw_b):
    cid = lax.axis_index("c")
    sid = lax.axis_index("s")
    wid = cid * NS + sid
    base = wid * EPW

    pltpu.sync_copy(src_hbm.at[pl.ds(base, EPW)], src_v)
    pltpu.sync_copy(dst_hbm.at[pl.ds(base, EPW)], dst_v)

    @pl.when(wid == 0)
    def _stage_acc():
        pltpu.sync_copy(accsum_hbm, acc_v.at[pl.ds(0, N_NODES)])

    slots = ((r1a, r2a, oa, semg_a, semw_a),
             (r1b, r2b, ob, semg_b, semw_b))

    def issue_gather(c, slot):
        r1, r2, _, semg, _ = slot
        off = c * CHUNK
        pltpu.async_copy(y1_hbm.at[src_v.at[pl.ds(off, CHUNK)]], r1, semg)
        pltpu.async_copy(y2_hbm.at[dst_v.at[pl.ds(off, CHUNK)]], r2, semg)

    def do_chunk(c, slot, nslot, issue_next, drain_write):
        r1, r2, o, semg, semw = slot
        if issue_next:
            issue_gather(c + 1, nslot)

        off = c * CHUNK
        pltpu.make_async_copy(
            y1_hbm.at[src_v.at[pl.ds(off, CHUNK)]], r1, semg).wait()
        pltpu.make_async_copy(
            y2_hbm.at[dst_v.at[pl.ds(off, CHUNK)]], r2, semg).wait()

        if drain_write:
            pltpu.make_async_copy(o, out_hbm.at[pl.ds(0, CHUNK)], semw).wait()

        @plsc.parallel_loop(0, CHUNK, unroll=4)
        def row_body(i):
            for k in range(D // (2 * L)):
                w1 = plsc.bitcast(r1[i, pl.ds(k * L, L)], jnp.bfloat16)
                w2 = plsc.bitcast(r2[i, pl.ds(k * L, L)], jnp.bfloat16)
                a1, b1 = plsc.unpack(w1, format=plsc.PackFormat.INTERLEAVED)
                a2, b2 = plsc.unpack(w2, format=plsc.PackFormat.INTERLEAVED)
                o[i, pl.ds(k * 2 * L, L)] = jnp.maximum(a1 + a2, 0.0)
                o[i, pl.ds(k * 2 * L + L, L)] = jnp.maximum(b1 + b2, 0.0)

        @pl.when(wid == 0)
        def _fix_col0():
            for j in range((CHUNK + L - 1) // L):
                valid = min(CHUNK - j * L, L)
                rows = lax.iota(jnp.int32, L) + j * L
                vals = acc_v[pl.ds(c * CHUNK + j * L, L)]
                if valid == L:
                    mask = None
                else:
                    mask = (lax.iota(jnp.int32, L)
                            < jnp.full((L,), valid, jnp.int32))
                plsc.addupdate_scatter(
                    o, [rows, jnp.zeros((L,), jnp.int32)], vals, mask=mask
                )

        pltpu.async_copy(o, out_hbm.at[pl.ds(base + c * CHUNK, CHUNK)], semw)

    issue_gather(0, slots[0])
    do_chunk(0, slots[0], slots[1], True, False)
    do_chunk(1, slots[1], slots[0], True, False)

    def group(g, carry):
        do_chunk(2 * g, slots[0], slots[1], True, True)
        do_chunk(2 * g + 1, slots[1], slots[0], True, True)
        return carry
    lax.fori_loop(1, (NCHUNK - 1) // 2, group, 0)

    do_chunk(NCHUNK - 1, slots[0], slots[1], False, True)

    pltpu.make_async_copy(oa, out_hbm.at[pl.ds(0, CHUNK)], semw_a).wait()
    pltpu.make_async_copy(ob, out_hbm.at[pl.ds(0, CHUNK)], semw_b).wait()


_edge_call = pl.kernel(
    _edge_body,
    out_type=jax.ShapeDtypeStruct((N_EDGES, D), jnp.float32),
    mesh=_mesh,
    scratch_types=[
        pltpu.VMEM((EPW,), jnp.int32),
        pltpu.VMEM((EPW,), jnp.int32),
        pltpu.VMEM((N_NODES + L,), jnp.float32),
        pltpu.VMEM((CHUNK, D // 2), jnp.int32),
        pltpu.VMEM((CHUNK, D // 2), jnp.int32),
        pltpu.VMEM((CHUNK, D), jnp.float32),
        pltpu.VMEM((CHUNK, D // 2), jnp.int32),
        pltpu.VMEM((CHUNK, D // 2), jnp.int32),
        pltpu.VMEM((CHUNK, D), jnp.float32),
        pltpu.SemaphoreType.DMA,
        pltpu.SemaphoreType.DMA,
        pltpu.SemaphoreType.DMA,
        pltpu.SemaphoreType.DMA,
    ],
    compiler_params=pltpu.CompilerParams(
        needs_layout_passes=False, use_tc_tiling_on_sc=False
    ),
)


def _to_interleaved_bf16(y):
    yp = y.reshape(N_NODES, D // (2 * L), 2, L).transpose(0, 1, 3, 2)
    yb = yp.reshape(N_NODES, D // 2, 2).astype(jnp.bfloat16)
    return lax.bitcast_convert_type(yb, jnp.int32)


def kernel(x, edge_index, W, b):
    src = edge_index[0]
    dst = edge_index[1]
    y1, y2 = _matmuls(x, W, b)
    c1 = y1[:, 0]
    c2 = y2[:, 0]
    acc2 = _acc_call(c1, c2, src, dst)
    accsum = acc2[0] + acc2[1]
    return _edge_call(_to_interleaved_bf16(y1), _to_interleaved_bf16(y2),
                      src, dst, accsum)
